# baseline (speedup 1.0000x reference)
import jax
import jax.numpy as jnp
from jax.experimental import pallas as pl
from jax.experimental.pallas import tpu as pltpu

_D = 128
_L = 8192
_CH = 2048
_NBUF = 8
_NCH = 16


def _stream_kernel(x_hbm, p_hbm, w_ref, b_ref, o_hbm,
                   xbuf, ybuf, pvm, in_sems, out_sems, p_sem):
    def in_copy(c, slot):
        return pltpu.make_async_copy(
            x_hbm.at[pl.ds(c * _CH, _CH), :], xbuf.at[slot], in_sems.at[slot])

    def out_copy(c, slot):
        return pltpu.make_async_copy(
            ybuf.at[slot], o_hbm.at[pl.ds(c * _CH, _CH), :], out_sems.at[slot])

    pcp = pltpu.make_async_copy(p_hbm, pvm, p_sem)
    pcp.start()
    for k in range(_NBUF):
        in_copy(k, k).start()
    pcp.wait()
    pvm[...] = pvm[...] + b_ref[...]

    for c in range(_NCH):
        slot = c % _NBUF
        if c >= _NBUF:
            out_copy(c - _NBUF, slot).wait()
        in_copy(c, slot).wait()
        pr = (c * _CH) % _L
        ybuf[slot] = (
            jnp.dot(xbuf[slot], w_ref[...], preferred_element_type=jnp.float32)
            + pvm[pl.ds(pr, _CH), :]
        )
        out_copy(c, slot).start()
        if c + _NBUF < _NCH:
            in_copy(c + _NBUF, slot).start()

    for c in range(_NCH - _NBUF, _NCH):
        out_copy(c, c % _NBUF).wait()


def kernel(inputs, pos_table, W, b):
    B, L, Din = inputs.shape
    Dout = W.shape[1]
    x2 = inputs.reshape(B * L, Din)
    b2 = b.reshape(1, Dout)
    out = pl.pallas_call(
        _stream_kernel,
        in_specs=[
            pl.BlockSpec(memory_space=pltpu.MemorySpace.HBM),
            pl.BlockSpec(memory_space=pltpu.MemorySpace.HBM),
            pl.BlockSpec(memory_space=pltpu.MemorySpace.VMEM),
            pl.BlockSpec(memory_space=pltpu.MemorySpace.VMEM),
        ],
        out_specs=pl.BlockSpec(memory_space=pltpu.MemorySpace.HBM),
        out_shape=jax.ShapeDtypeStruct((B * L, Dout), jnp.float32),
        scratch_shapes=[
            pltpu.VMEM((_NBUF, _CH, Din), jnp.float32),
            pltpu.VMEM((_NBUF, _CH, Dout), jnp.float32),
            pltpu.VMEM((_L, Dout), jnp.float32),
            pltpu.SemaphoreType.DMA((_NBUF,)),
            pltpu.SemaphoreType.DMA((_NBUF,)),
            pltpu.SemaphoreType.DMA,
        ],
        compiler_params=pltpu.CompilerParams(
            vmem_limit_bytes=100 * 1024 * 1024,
        ),
    )(x2, pos_table, W, b2)
    return out.reshape(B, L, Dout)
